# Initial kernel scaffold; baseline (speedup 1.0000x reference)
import functools

import jax
import jax.numpy as jnp
from jax import lax
from jax.experimental import pallas as pl
from jax.experimental.pallas import tpu as pltpu
from jax.experimental.pallas import tpu_sc as plsc

N = 10000
E = 320000
D_IN = 128
H1 = 128
H2 = 64
C = 10

NC = 2
NS = 16
NW = NC * NS
EPW = E // NW
B = 80
NCHUNK = EPW // B
SLAB = 640
LAST_SLAB = N - (NS - 1) * SLAB
assert E % NW == 0 and EPW % B == 0 and LAST_SLAB > 0 and LAST_SLAB % B == 0

_MESH = plsc.VectorSubcoreMesh(
    core_axis_name="c", subcore_axis_name="s", num_cores=NC, num_subcores=NS
)


@functools.partial(
    pl.kernel,
    out_type=jax.ShapeDtypeStruct((NC, N), jnp.float32),
    mesh=_MESH,
    scratch_types=[
        pltpu.VMEM((B,), jnp.int32),
        pltpu.VMEM((B,), jnp.float32),
        pltpu.VMEM((SLAB,), jnp.float32),
        pltpu.VMEM_SHARED((N,), jnp.float32),
    ],
)
def _deg_kernel(dst_hbm, degp_hbm, dst_v, ones_v, zeros_v, acc):
    cid = lax.axis_index("c")
    sid = lax.axis_index("s")
    wid = sid * NC + cid
    one16 = jnp.ones((16,), jnp.float32)
    zero16 = jnp.zeros((16,), jnp.float32)
    for i in range(B // 16):
        ones_v[pl.ds(i * 16, 16)] = one16
    for i in range(SLAB // 16):
        zeros_v[pl.ds(i * 16, 16)] = zero16

    @pl.when(sid < NS - 1)
    def _():
        pltpu.sync_copy(zeros_v, acc.at[pl.ds(sid * SLAB, SLAB)])

    @pl.when(sid == NS - 1)
    def _():
        pltpu.sync_copy(zeros_v.at[pl.ds(0, LAST_SLAB)],
                        acc.at[pl.ds(sid * SLAB, LAST_SLAB)])

    plsc.subcore_barrier()
    base = wid * EPW

    def body(c, carry):
        pltpu.sync_copy(dst_hbm.at[pl.ds(base + c * B, B)], dst_v)
        pltpu.sync_copy(ones_v, acc.at[dst_v], add=True)
        return carry

    lax.fori_loop(0, NCHUNK, body, 0)
    plsc.subcore_barrier()

    @pl.when(sid < NS - 1)
    def _():
        pltpu.sync_copy(acc.at[pl.ds(sid * SLAB, SLAB)],
                        degp_hbm.at[cid, pl.ds(sid * SLAB, SLAB)])

    @pl.when(sid == NS - 1)
    def _():
        pltpu.sync_copy(acc.at[pl.ds(sid * SLAB, LAST_SLAB)],
                        degp_hbm.at[cid, pl.ds(sid * SLAB, LAST_SLAB)])


def _make_agg(D):
    @functools.partial(
        pl.kernel,
        out_type=jax.ShapeDtypeStruct((NC, N, D), jnp.float32),
        mesh=_MESH,
        scratch_types=[
            pltpu.VMEM((B,), jnp.int32),
            pltpu.VMEM((B,), jnp.int32),
            pltpu.VMEM((B, D), jnp.float32),
            pltpu.VMEM_SHARED((N, D), jnp.float32),
            pltpu.SemaphoreType.DMA,
        ],
    )
    def agg(xs_hbm, src_hbm, dst_hbm, parts_hbm, src_v, dst_v, rows_v, acc, sem):
        cid = lax.axis_index("c")
        sid = lax.axis_index("s")
        wid = sid * NC + cid
        zero16 = jnp.zeros((16,), jnp.float32)

        def zrow(r, carry):
            for j in range(D // 16):
                rows_v[r, pl.ds(j * 16, 16)] = zero16
            return carry

        lax.fori_loop(0, B, zrow, 0)
        for b in range(SLAB // B):
            if b < LAST_SLAB // B:
                pltpu.sync_copy(rows_v, acc.at[pl.ds(sid * SLAB + b * B, B)])
            else:
                @pl.when(sid < NS - 1)
                def _():
                    pltpu.sync_copy(rows_v, acc.at[pl.ds(sid * SLAB + b * B, B)])

        plsc.subcore_barrier()
        base = wid * EPW

        def body(c, carry):
            pltpu.sync_copy(src_hbm.at[pl.ds(base + c * B, B)], src_v)
            pltpu.sync_copy(dst_hbm.at[pl.ds(base + c * B, B)], dst_v)
            pltpu.async_copy(xs_hbm.at[src_v], rows_v, sem).wait()
            pltpu.sync_copy(rows_v, acc.at[dst_v], add=True)
            return carry

        lax.fori_loop(0, NCHUNK, body, 0)
        plsc.subcore_barrier()

        @pl.when(sid < NS - 1)
        def _():
            pltpu.sync_copy(acc.at[pl.ds(sid * SLAB, SLAB)],
                            parts_hbm.at[cid, pl.ds(sid * SLAB, SLAB)])

        @pl.when(sid == NS - 1)
        def _():
            pltpu.sync_copy(acc.at[pl.ds(sid * SLAB, LAST_SLAB)],
                            parts_hbm.at[cid, pl.ds(sid * SLAB, LAST_SLAB)])

    return agg


_agg_h1 = _make_agg(H1)
_agg_h2 = _make_agg(H2)


R = 1000
G = N // R


def _tc1_body(x_ref, w_ref, degT_ref, xw_ref, xs_ref, dinv_ref):
    deg = degT_ref[:, 0] + degT_ref[:, 1] + 1.0
    dv = lax.rsqrt(deg)
    dinv_ref[...] = dv[:, None]
    xw = jnp.dot(x_ref[...], w_ref[...], preferred_element_type=jnp.float32)
    xw_ref[...] = xw
    xs_ref[...] = xw * dv[:, None]


def _tc1(x, w1, degT):
    return pl.pallas_call(
        _tc1_body,
        grid=(G,),
        in_specs=[
            pl.BlockSpec((R, D_IN), lambda i: (i, 0)),
            pl.BlockSpec((D_IN, H1), lambda i: (0, 0)),
            pl.BlockSpec((R, NC), lambda i: (i, 0)),
        ],
        out_specs=[
            pl.BlockSpec((R, H1), lambda i: (i, 0)),
            pl.BlockSpec((R, H1), lambda i: (i, 0)),
            pl.BlockSpec((R, 1), lambda i: (i, 0)),
        ],
        out_shape=[
            jax.ShapeDtypeStruct((N, H1), jnp.float32),
            jax.ShapeDtypeStruct((N, H1), jnp.float32),
            jax.ShapeDtypeStruct((N, 1), jnp.float32),
        ],
    )(x, w1, degT)


def _tc2_body(p_ref, dv_ref, xw_ref, b1_ref, w2_ref, hw_ref, xs2_ref):
    agg = p_ref[0] + p_ref[1]
    dv = dv_ref[...]
    h = jnp.maximum(agg * dv + xw_ref[...] * (dv * dv) + b1_ref[...], 0.0)
    hw = jnp.dot(h, w2_ref[...], preferred_element_type=jnp.float32)
    hw_ref[...] = hw
    xs2_ref[...] = hw * dv


def _tc2(parts, dinv, xw, b1, w2):
    return pl.pallas_call(
        _tc2_body,
        grid=(G,),
        in_specs=[
            pl.BlockSpec((NC, R, H1), lambda i: (0, i, 0)),
            pl.BlockSpec((R, 1), lambda i: (i, 0)),
            pl.BlockSpec((R, H1), lambda i: (i, 0)),
            pl.BlockSpec((1, H1), lambda i: (0, 0)),
            pl.BlockSpec((H1, H2), lambda i: (0, 0)),
        ],
        out_specs=[
            pl.BlockSpec((R, H2), lambda i: (i, 0)),
            pl.BlockSpec((R, H2), lambda i: (i, 0)),
        ],
        out_shape=[
            jax.ShapeDtypeStruct((N, H2), jnp.float32),
            jax.ShapeDtypeStruct((N, H2), jnp.float32),
        ],
    )(parts, dinv, xw, b1, w2)


def _tc3_body(p_ref, dv_ref, hw_ref, b2_ref, wc_ref, bc_ref, out_ref):
    agg = p_ref[0] + p_ref[1]
    dv = dv_ref[...]
    h = jnp.maximum(agg * dv + hw_ref[...] * (dv * dv) + b2_ref[...], 0.0)
    out_ref[...] = (
        jnp.dot(h, wc_ref[...], preferred_element_type=jnp.float32) + bc_ref[...]
    )


def _tc3(parts, dinv, hw, b2, wc, bc):
    return pl.pallas_call(
        _tc3_body,
        grid=(G,),
        in_specs=[
            pl.BlockSpec((NC, R, H2), lambda i: (0, i, 0)),
            pl.BlockSpec((R, 1), lambda i: (i, 0)),
            pl.BlockSpec((R, H2), lambda i: (i, 0)),
            pl.BlockSpec((1, H2), lambda i: (0, 0)),
            pl.BlockSpec((H2, C), lambda i: (0, 0)),
            pl.BlockSpec((1, C), lambda i: (0, 0)),
        ],
        out_specs=pl.BlockSpec((R, C), lambda i: (i, 0)),
        out_shape=jax.ShapeDtypeStruct((N, C), jnp.float32),
    )(parts, dinv, hw, b2, wc, bc)


def kernel(x, edge_index, W1, b1, W2, b2, Wc, bc):
    src = edge_index[0].astype(jnp.int32)
    dst = edge_index[1].astype(jnp.int32)
    degp = _deg_kernel(dst)
    xw, xs, dinv = _tc1(x, W1, degp.T)
    parts1 = _agg_h1(xs, src, dst)
    hw, xs2 = _tc2(parts1, dinv, xw, b1.reshape(1, H1), W2)
    parts2 = _agg_h2(xs2, src, dst)
    return _tc3(parts2, dinv, hw, b2.reshape(1, H2), Wc, bc.reshape(1, C))


# speedup vs baseline: 13.5310x; 13.5310x over previous
import functools

import jax
import jax.numpy as jnp
from jax import lax
from jax.experimental import pallas as pl
from jax.experimental.pallas import tpu as pltpu
from jax.experimental.pallas import tpu_sc as plsc

N = 10000
E = 320000
D_IN = 128
H1 = 128
H2 = 64
C = 10

NC = 2
NS = 16
NW = NC * NS
EPW = E // NW
B = 80
NCHUNK = EPW // B
SLAB = 640
LAST_SLAB = N - (NS - 1) * SLAB
NPAD = NS * SLAB
assert E % NW == 0 and EPW % B == 0 and LAST_SLAB > 0 and LAST_SLAB % B == 0

_MESH = plsc.VectorSubcoreMesh(
    core_axis_name="c", subcore_axis_name="s", num_cores=NC, num_subcores=NS
)


@functools.partial(
    pl.kernel,
    out_type=jax.ShapeDtypeStruct((NC, 1, NPAD), jnp.float32),
    mesh=_MESH,
    scratch_types=[
        pltpu.VMEM((B,), jnp.int32),
        pltpu.VMEM((B,), jnp.float32),
        pltpu.VMEM((SLAB,), jnp.float32),
        pltpu.VMEM_SHARED((NPAD,), jnp.float32),
    ],
)
def _deg_kernel(dst_hbm, degp_hbm, dst_v, ones_v, zeros_v, acc):
    cid = lax.axis_index("c")
    sid = lax.axis_index("s")
    wid = sid * NC + cid
    one16 = jnp.ones((16,), jnp.float32)
    zero16 = jnp.zeros((16,), jnp.float32)
    for i in range(B // 16):
        ones_v[pl.ds(i * 16, 16)] = one16
    for i in range(SLAB // 16):
        zeros_v[pl.ds(i * 16, 16)] = zero16

    pltpu.sync_copy(zeros_v, acc.at[pl.ds(sid * SLAB, SLAB)])
    plsc.subcore_barrier()
    base = wid * EPW

    def body(c, carry):
        pltpu.sync_copy(dst_hbm.at[pl.ds(base + c * B, B)], dst_v)
        pltpu.sync_copy(ones_v, acc.at[dst_v], add=True)
        return carry

    lax.fori_loop(0, NCHUNK, body, 0)
    plsc.subcore_barrier()
    pltpu.sync_copy(acc.at[pl.ds(sid * SLAB, SLAB)],
                    degp_hbm.at[cid, 0, pl.ds(sid * SLAB, SLAB)])


def _make_agg(D):
    @functools.partial(
        pl.kernel,
        out_type=jax.ShapeDtypeStruct((NC, N, D), jnp.float32),
        mesh=_MESH,
        scratch_types=[
            pltpu.VMEM((B,), jnp.int32),
            pltpu.VMEM((B,), jnp.int32),
            pltpu.VMEM((B, D), jnp.float32),
            pltpu.VMEM_SHARED((N, D), jnp.float32),
            pltpu.SemaphoreType.DMA,
        ],
    )
    def agg(xs_hbm, src_hbm, dst_hbm, parts_hbm, src_v, dst_v, rows_v, acc, sem):
        cid = lax.axis_index("c")
        sid = lax.axis_index("s")
        wid = sid * NC + cid
        zero16 = jnp.zeros((16,), jnp.float32)

        def zrow(r, carry):
            for j in range(D // 16):
                rows_v[r, pl.ds(j * 16, 16)] = zero16
            return carry

        lax.fori_loop(0, B, zrow, 0)
        for b in range(SLAB // B):
            if b < LAST_SLAB // B:
                pltpu.sync_copy(rows_v, acc.at[pl.ds(sid * SLAB + b * B, B)])
            else:
                @pl.when(sid < NS - 1)
                def _():
                    pltpu.sync_copy(rows_v, acc.at[pl.ds(sid * SLAB + b * B, B)])

        plsc.subcore_barrier()
        base = wid * EPW

        def body(c, carry):
            pltpu.sync_copy(src_hbm.at[pl.ds(base + c * B, B)], src_v)
            pltpu.sync_copy(dst_hbm.at[pl.ds(base + c * B, B)], dst_v)
            pltpu.async_copy(xs_hbm.at[src_v], rows_v, sem).wait()
            pltpu.sync_copy(rows_v, acc.at[dst_v], add=True)
            return carry

        lax.fori_loop(0, NCHUNK, body, 0)
        plsc.subcore_barrier()

        @pl.when(sid < NS - 1)
        def _():
            pltpu.sync_copy(acc.at[pl.ds(sid * SLAB, SLAB)],
                            parts_hbm.at[cid, pl.ds(sid * SLAB, SLAB)])

        @pl.when(sid == NS - 1)
        def _():
            pltpu.sync_copy(acc.at[pl.ds(sid * SLAB, LAST_SLAB)],
                            parts_hbm.at[cid, pl.ds(sid * SLAB, LAST_SLAB)])

    return agg


_agg128 = _make_agg(H1)


R = 1000
G = N // R


def _tc1_body(x_ref, w_ref, degT_ref, xw_ref, xs_ref, dinv_ref):
    deg = degT_ref[:, 0] + degT_ref[:, 1] + 1.0
    dv = lax.rsqrt(deg)
    dinv_ref[...] = dv[:, None]
    xw = jnp.dot(x_ref[...], w_ref[...], preferred_element_type=jnp.float32)
    xw_ref[...] = xw
    xs_ref[...] = xw * dv[:, None]


def _tc1(x, w1, degT):
    return pl.pallas_call(
        _tc1_body,
        grid=(G,),
        in_specs=[
            pl.BlockSpec((R, D_IN), lambda i: (i, 0)),
            pl.BlockSpec((D_IN, H1), lambda i: (0, 0)),
            pl.BlockSpec((R, NC), lambda i: (i, 0)),
        ],
        out_specs=[
            pl.BlockSpec((R, H1), lambda i: (i, 0)),
            pl.BlockSpec((R, H1), lambda i: (i, 0)),
            pl.BlockSpec((R, 1), lambda i: (i, 0)),
        ],
        out_shape=[
            jax.ShapeDtypeStruct((N, H1), jnp.float32),
            jax.ShapeDtypeStruct((N, H1), jnp.float32),
            jax.ShapeDtypeStruct((N, 1), jnp.float32),
        ],
    )(x, w1, degT)


def _tc2_body(p_ref, dv_ref, xw_ref, b1_ref, w2_ref, hw_ref, xs2_ref):
    agg = p_ref[0] + p_ref[1]
    dv = dv_ref[...]
    h = jnp.maximum(agg * dv + xw_ref[...] * (dv * dv) + b1_ref[...], 0.0)
    hw = jnp.dot(h, w2_ref[...], preferred_element_type=jnp.float32)
    hw_ref[...] = hw
    xs2_ref[...] = jnp.concatenate(
        [hw * dv, jnp.zeros((R, H1 - H2), jnp.float32)], axis=1
    )


def _tc2(parts, dinv, xw, b1, w2):
    return pl.pallas_call(
        _tc2_body,
        grid=(G,),
        in_specs=[
            pl.BlockSpec((NC, R, H1), lambda i: (0, i, 0)),
            pl.BlockSpec((R, 1), lambda i: (i, 0)),
            pl.BlockSpec((R, H1), lambda i: (i, 0)),
            pl.BlockSpec((1, H1), lambda i: (0, 0)),
            pl.BlockSpec((H1, H2), lambda i: (0, 0)),
        ],
        out_specs=[
            pl.BlockSpec((R, H2), lambda i: (i, 0)),
            pl.BlockSpec((R, H1), lambda i: (i, 0)),
        ],
        out_shape=[
            jax.ShapeDtypeStruct((N, H2), jnp.float32),
            jax.ShapeDtypeStruct((N, H1), jnp.float32),
        ],
    )(parts, dinv, xw, b1, w2)


def _tc3_body(p_ref, dv_ref, hw_ref, b2_ref, wc_ref, bc_ref, out_ref):
    agg = (p_ref[0] + p_ref[1])[:, :H2]
    dv = dv_ref[...]
    h = jnp.maximum(agg * dv + hw_ref[...] * (dv * dv) + b2_ref[...], 0.0)
    out_ref[...] = (
        jnp.dot(h, wc_ref[...], preferred_element_type=jnp.float32) + bc_ref[...]
    )


def _tc3(parts, dinv, hw, b2, wc, bc):
    return pl.pallas_call(
        _tc3_body,
        grid=(G,),
        in_specs=[
            pl.BlockSpec((NC, R, H1), lambda i: (0, i, 0)),
            pl.BlockSpec((R, 1), lambda i: (i, 0)),
            pl.BlockSpec((R, H2), lambda i: (i, 0)),
            pl.BlockSpec((1, H2), lambda i: (0, 0)),
            pl.BlockSpec((H2, C), lambda i: (0, 0)),
            pl.BlockSpec((1, C), lambda i: (0, 0)),
        ],
        out_specs=pl.BlockSpec((R, C), lambda i: (i, 0)),
        out_shape=jax.ShapeDtypeStruct((N, C), jnp.float32),
    )(parts, dinv, hw, b2, wc, bc)


def kernel(x, edge_index, W1, b1, W2, b2, Wc, bc):
    src = edge_index[0].astype(jnp.int32)
    dst = edge_index[1].astype(jnp.int32)
    degp = _deg_kernel(dst)
    xw, xs, dinv = _tc1(x, W1, degp[:, 0, :N].T)
    parts1 = _agg128(xs, src, dst)
    hw, xs2 = _tc2(parts1, dinv, xw, b1.reshape(1, H1), W2)
    parts2 = _agg128(xs2, src, dst)
    return _tc3(parts2, dinv, hw, b2.reshape(1, H2), Wc, bc.reshape(1, C))


# speedup vs baseline: 25.7901x; 1.9060x over previous
import functools

import jax
import jax.numpy as jnp
from jax import lax
from jax.experimental import pallas as pl
from jax.experimental.pallas import tpu as pltpu
from jax.experimental.pallas import tpu_sc as plsc

N = 10000
E = 320000
D_IN = 128
H1 = 128
H2 = 64
C = 10

NC = 2
NS = 16
NW = NC * NS
EPW = E // NW
B = 80
NCHUNK = EPW // B
SLAB = 640
LAST_SLAB = N - (NS - 1) * SLAB
NPAD = NS * SLAB
assert E % NW == 0 and EPW % B == 0 and LAST_SLAB > 0 and LAST_SLAB % B == 0
assert NCHUNK % 2 == 1

_MESH = plsc.VectorSubcoreMesh(
    core_axis_name="c", subcore_axis_name="s", num_cores=NC, num_subcores=NS
)


DEG_DEPTH = 4


@functools.partial(
    pl.kernel,
    out_type=jax.ShapeDtypeStruct((NC, 1, NPAD), jnp.float32),
    mesh=_MESH,
    scratch_types=[
        pltpu.VMEM((NCHUNK, B), jnp.int32),
        pltpu.VMEM((B,), jnp.float32),
        pltpu.VMEM((SLAB,), jnp.float32),
        pltpu.VMEM_SHARED((NPAD,), jnp.float32),
        pltpu.SemaphoreType.DMA,
    ],
)
def _deg_kernel(dst3_hbm, degp_hbm, dsts, ones_v, zeros_v, acc, sem):
    cid = lax.axis_index("c")
    sid = lax.axis_index("s")
    wid = sid * NC + cid
    one16 = jnp.ones((16,), jnp.float32)
    zero16 = jnp.zeros((16,), jnp.float32)
    for i in range(B // 16):
        ones_v[pl.ds(i * 16, 16)] = one16
    for i in range(SLAB // 16):
        zeros_v[pl.ds(i * 16, 16)] = zero16

    pltpu.sync_copy(zeros_v, acc.at[pl.ds(sid * SLAB, SLAB)])
    pltpu.sync_copy(dst3_hbm.at[wid], dsts)
    plsc.subcore_barrier()

    for k in range(DEG_DEPTH):
        pltpu.async_copy(ones_v, acc.at[dsts.at[k]], sem, add=True)

    def body(c, carry):
        pltpu.make_async_copy(ones_v, acc.at[dsts.at[0]], sem).wait()
        pltpu.async_copy(ones_v, acc.at[dsts.at[c]], sem, add=True)
        return carry

    lax.fori_loop(DEG_DEPTH, NCHUNK, body, 0)
    for k in range(DEG_DEPTH):
        pltpu.make_async_copy(ones_v, acc.at[dsts.at[0]], sem).wait()
    plsc.subcore_barrier()
    pltpu.sync_copy(acc.at[pl.ds(sid * SLAB, SLAB)],
                    degp_hbm.at[cid, 0, pl.ds(sid * SLAB, SLAB)])


def _make_agg(D):
    @functools.partial(
        pl.kernel,
        out_type=jax.ShapeDtypeStruct((NC, N, D), jnp.float32),
        mesh=_MESH,
        scratch_types=[
            pltpu.VMEM((EPW,), jnp.int32),
            pltpu.VMEM((NCHUNK, B), jnp.int32),
            pltpu.VMEM((B, D), jnp.float32),
            pltpu.VMEM((B, D), jnp.float32),
            pltpu.VMEM_SHARED((N, D), jnp.float32),
            pltpu.SemaphoreType.DMA,
            pltpu.SemaphoreType.DMA,
            pltpu.SemaphoreType.DMA,
            pltpu.SemaphoreType.DMA,
        ],
    )
    def agg(xs_hbm, src_hbm, dst3_hbm, parts_hbm,
            srcs, dsts, rows0, rows1, acc, gs0, gs1, ss0, ss1):
        cid = lax.axis_index("c")
        sid = lax.axis_index("s")
        wid = sid * NC + cid
        zero16 = jnp.zeros((16,), jnp.float32)

        def zrow(r, carry):
            for j in range(D // 16):
                rows0[r, pl.ds(j * 16, 16)] = zero16
            return carry

        lax.fori_loop(0, B, zrow, 0)
        for b in range(SLAB // B):
            if b < LAST_SLAB // B:
                pltpu.sync_copy(rows0, acc.at[pl.ds(sid * SLAB + b * B, B)])
            else:
                @pl.when(sid < NS - 1)
                def _():
                    pltpu.sync_copy(rows0, acc.at[pl.ds(sid * SLAB + b * B, B)])

        pltpu.sync_copy(src_hbm.at[pl.ds(wid * EPW, EPW)], srcs)
        pltpu.sync_copy(dst3_hbm.at[wid], dsts)
        plsc.subcore_barrier()

        def gidx(c):
            return srcs.at[pl.ds(c * B, B)]

        def wait_g(rows, sem):
            pltpu.make_async_copy(xs_hbm.at[gidx(0)], rows, sem).wait()

        def wait_s(rows, sem):
            pltpu.make_async_copy(rows, acc.at[dsts.at[0]], sem).wait()

        pltpu.async_copy(xs_hbm.at[gidx(0)], rows0, gs0)

        def body(i, carry):
            a = 2 * i
            wait_g(rows0, gs0)

            @pl.when(i > 0)
            def _():
                wait_s(rows1, ss1)
            pltpu.async_copy(xs_hbm.at[gidx(a + 1)], rows1, gs1)
            pltpu.async_copy(rows0, acc.at[dsts.at[a]], ss0, add=True)
            wait_g(rows1, gs1)
            wait_s(rows0, ss0)
            pltpu.async_copy(xs_hbm.at[gidx(a + 2)], rows0, gs0)
            pltpu.async_copy(rows1, acc.at[dsts.at[a + 1]], ss1, add=True)
            return carry

        lax.fori_loop(0, NCHUNK // 2, body, 0)
        wait_g(rows0, gs0)
        pltpu.async_copy(rows0, acc.at[dsts.at[NCHUNK - 1]], ss0, add=True)
        wait_s(rows0, ss0)
        wait_s(rows1, ss1)
        plsc.subcore_barrier()

        @pl.when(sid < NS - 1)
        def _():
            pltpu.sync_copy(acc.at[pl.ds(sid * SLAB, SLAB)],
                            parts_hbm.at[cid, pl.ds(sid * SLAB, SLAB)])

        @pl.when(sid == NS - 1)
        def _():
            pltpu.sync_copy(acc.at[pl.ds(sid * SLAB, LAST_SLAB)],
                            parts_hbm.at[cid, pl.ds(sid * SLAB, LAST_SLAB)])

    return agg


_agg128 = _make_agg(H1)


R = 1000
G = N // R


def _tc1_body(x_ref, w_ref, degT_ref, xw_ref, xs_ref, dinv_ref):
    deg = degT_ref[:, 0] + degT_ref[:, 1] + 1.0
    dv = lax.rsqrt(deg)
    dinv_ref[...] = dv[:, None]
    xw = jnp.dot(x_ref[...], w_ref[...], preferred_element_type=jnp.float32)
    xw_ref[...] = xw
    xs_ref[...] = xw * dv[:, None]


def _tc1(x, w1, degT):
    return pl.pallas_call(
        _tc1_body,
        grid=(G,),
        in_specs=[
            pl.BlockSpec((R, D_IN), lambda i: (i, 0)),
            pl.BlockSpec((D_IN, H1), lambda i: (0, 0)),
            pl.BlockSpec((R, NC), lambda i: (i, 0)),
        ],
        out_specs=[
            pl.BlockSpec((R, H1), lambda i: (i, 0)),
            pl.BlockSpec((R, H1), lambda i: (i, 0)),
            pl.BlockSpec((R, 1), lambda i: (i, 0)),
        ],
        out_shape=[
            jax.ShapeDtypeStruct((N, H1), jnp.float32),
            jax.ShapeDtypeStruct((N, H1), jnp.float32),
            jax.ShapeDtypeStruct((N, 1), jnp.float32),
        ],
    )(x, w1, degT)


def _tc2_body(p_ref, dv_ref, xw_ref, b1_ref, w2_ref, hw_ref, xs2_ref):
    agg = p_ref[0] + p_ref[1]
    dv = dv_ref[...]
    h = jnp.maximum(agg * dv + xw_ref[...] * (dv * dv) + b1_ref[...], 0.0)
    hw = jnp.dot(h, w2_ref[...], preferred_element_type=jnp.float32)
    hw_ref[...] = hw
    xs2_ref[...] = jnp.concatenate(
        [hw * dv, jnp.zeros((R, H1 - H2), jnp.float32)], axis=1
    )


def _tc2(parts, dinv, xw, b1, w2):
    return pl.pallas_call(
        _tc2_body,
        grid=(G,),
        in_specs=[
            pl.BlockSpec((NC, R, H1), lambda i: (0, i, 0)),
            pl.BlockSpec((R, 1), lambda i: (i, 0)),
            pl.BlockSpec((R, H1), lambda i: (i, 0)),
            pl.BlockSpec((1, H1), lambda i: (0, 0)),
            pl.BlockSpec((H1, H2), lambda i: (0, 0)),
        ],
        out_specs=[
            pl.BlockSpec((R, H2), lambda i: (i, 0)),
            pl.BlockSpec((R, H1), lambda i: (i, 0)),
        ],
        out_shape=[
            jax.ShapeDtypeStruct((N, H2), jnp.float32),
            jax.ShapeDtypeStruct((N, H1), jnp.float32),
        ],
    )(parts, dinv, xw, b1, w2)


def _tc3_body(p_ref, dv_ref, hw_ref, b2_ref, wc_ref, bc_ref, out_ref):
    agg = (p_ref[0] + p_ref[1])[:, :H2]
    dv = dv_ref[...]
    h = jnp.maximum(agg * dv + hw_ref[...] * (dv * dv) + b2_ref[...], 0.0)
    out_ref[...] = (
        jnp.dot(h, wc_ref[...], preferred_element_type=jnp.float32) + bc_ref[...]
    )


def _tc3(parts, dinv, hw, b2, wc, bc):
    return pl.pallas_call(
        _tc3_body,
        grid=(G,),
        in_specs=[
            pl.BlockSpec((NC, R, H1), lambda i: (0, i, 0)),
            pl.BlockSpec((R, 1), lambda i: (i, 0)),
            pl.BlockSpec((R, H2), lambda i: (i, 0)),
            pl.BlockSpec((1, H2), lambda i: (0, 0)),
            pl.BlockSpec((H2, C), lambda i: (0, 0)),
            pl.BlockSpec((1, C), lambda i: (0, 0)),
        ],
        out_specs=pl.BlockSpec((R, C), lambda i: (i, 0)),
        out_shape=jax.ShapeDtypeStruct((N, C), jnp.float32),
    )(parts, dinv, hw, b2, wc, bc)


def kernel(x, edge_index, W1, b1, W2, b2, Wc, bc):
    src = edge_index[0].astype(jnp.int32)
    dst3 = edge_index[1].astype(jnp.int32).reshape(NW, NCHUNK, B)
    degp = _deg_kernel(dst3)
    xw, xs, dinv = _tc1(x, W1, degp[:, 0, :N].T)
    parts1 = _agg128(xs, src, dst3)
    hw, xs2 = _tc2(parts1, dinv, xw, b1.reshape(1, H1), W2)
    parts2 = _agg128(xs2, src, dst3)
    return _tc3(parts2, dinv, hw, b2.reshape(1, H2), Wc, bc.reshape(1, C))


# speedup vs baseline: 25.8374x; 1.0018x over previous
import functools

import jax
import jax.numpy as jnp
from jax import lax
from jax.experimental import pallas as pl
from jax.experimental.pallas import tpu as pltpu
from jax.experimental.pallas import tpu_sc as plsc

N = 10000
E = 320000
D_IN = 128
H1 = 128
H2 = 64
C = 10

NC = 2
NS = 16
NW = NC * NS
EPW = E // NW
B = 80
NCHUNK = EPW // B
SLAB = 640
LAST_SLAB = N - (NS - 1) * SLAB
NPAD = NS * SLAB
assert E % NW == 0 and EPW % B == 0 and LAST_SLAB > 0 and LAST_SLAB % B == 0
assert NCHUNK % 2 == 1

_MESH = plsc.VectorSubcoreMesh(
    core_axis_name="c", subcore_axis_name="s", num_cores=NC, num_subcores=NS
)


DEG_DEPTH = 4


@functools.partial(
    pl.kernel,
    out_type=jax.ShapeDtypeStruct((NC, 1, NPAD), jnp.float32),
    mesh=_MESH,
    scratch_types=[
        pltpu.VMEM((NCHUNK, B), jnp.int32),
        pltpu.VMEM((B,), jnp.float32),
        pltpu.VMEM((SLAB,), jnp.float32),
        pltpu.VMEM_SHARED((NPAD,), jnp.float32),
        pltpu.SemaphoreType.DMA,
    ],
)
def _deg_kernel(dst3_hbm, degp_hbm, dsts, ones_v, zeros_v, acc, sem):
    cid = lax.axis_index("c")
    sid = lax.axis_index("s")
    wid = sid * NC + cid
    one16 = jnp.ones((16,), jnp.float32)
    zero16 = jnp.zeros((16,), jnp.float32)
    for i in range(B // 16):
        ones_v[pl.ds(i * 16, 16)] = one16
    for i in range(SLAB // 16):
        zeros_v[pl.ds(i * 16, 16)] = zero16

    pltpu.sync_copy(zeros_v, acc.at[pl.ds(sid * SLAB, SLAB)])
    pltpu.sync_copy(dst3_hbm.at[wid], dsts)
    plsc.subcore_barrier()

    for k in range(DEG_DEPTH):
        pltpu.async_copy(ones_v, acc.at[dsts.at[k]], sem, add=True)

    def body(c, carry):
        pltpu.make_async_copy(ones_v, acc.at[dsts.at[0]], sem).wait()
        pltpu.async_copy(ones_v, acc.at[dsts.at[c]], sem, add=True)
        return carry

    lax.fori_loop(DEG_DEPTH, NCHUNK, body, 0)
    for k in range(DEG_DEPTH):
        pltpu.make_async_copy(ones_v, acc.at[dsts.at[0]], sem).wait()
    plsc.subcore_barrier()
    pltpu.sync_copy(acc.at[pl.ds(sid * SLAB, SLAB)],
                    degp_hbm.at[cid, 0, pl.ds(sid * SLAB, SLAB)])


def _make_agg(D):
    @functools.partial(
        pl.kernel,
        out_type=jax.ShapeDtypeStruct((NC, N, D), jnp.float32),
        mesh=_MESH,
        scratch_types=[
            pltpu.VMEM((EPW,), jnp.int32),
            pltpu.VMEM((NCHUNK, B), jnp.int32),
            pltpu.VMEM((B, D), jnp.float32),
            pltpu.VMEM((B, D), jnp.float32),
            pltpu.VMEM_SHARED((N, D), jnp.float32),
            pltpu.SemaphoreType.DMA,
            pltpu.SemaphoreType.DMA,
            pltpu.SemaphoreType.DMA,
            pltpu.SemaphoreType.DMA,
        ],
    )
    def agg(xs_hbm, src_hbm, dst3_hbm, parts_hbm,
            srcs, dsts, rows0, rows1, acc, gs0, gs1, ss0, ss1):
        cid = lax.axis_index("c")
        sid = lax.axis_index("s")
        wid = sid * NC + cid
        zero16 = jnp.zeros((16,), jnp.float32)

        def zrow(r, carry):
            for j in range(D // 16):
                rows0[r, pl.ds(j * 16, 16)] = zero16
            return carry

        lax.fori_loop(0, B, zrow, 0)
        for b in range(SLAB // B):
            if b < LAST_SLAB // B:
                pltpu.sync_copy(rows0, acc.at[pl.ds(sid * SLAB + b * B, B)])
            else:
                @pl.when(sid < NS - 1)
                def _():
                    pltpu.sync_copy(rows0, acc.at[pl.ds(sid * SLAB + b * B, B)])

        pltpu.sync_copy(src_hbm.at[pl.ds(wid * EPW, EPW)], srcs)
        pltpu.sync_copy(dst3_hbm.at[wid], dsts)
        plsc.subcore_barrier()

        HB = B // 2

        def issue_g(c, rows, sem):
            for h in range(2):
                pltpu.async_copy(
                    xs_hbm.at[srcs.at[pl.ds(c * B + h * HB, HB)]],
                    rows.at[pl.ds(h * HB, HB)], sem)

        def wait_g(rows, sem):
            for _ in range(2):
                pltpu.make_async_copy(
                    xs_hbm.at[srcs.at[pl.ds(0, HB)]],
                    rows.at[pl.ds(0, HB)], sem).wait()

        def wait_s(rows, sem):
            pltpu.make_async_copy(rows, acc.at[dsts.at[0]], sem).wait()

        issue_g(0, rows0, gs0)

        def body(i, carry):
            a = 2 * i
            wait_g(rows0, gs0)

            @pl.when(i > 0)
            def _():
                wait_s(rows1, ss1)
            issue_g(a + 1, rows1, gs1)
            pltpu.async_copy(rows0, acc.at[dsts.at[a]], ss0, add=True)
            wait_g(rows1, gs1)
            wait_s(rows0, ss0)
            issue_g(a + 2, rows0, gs0)
            pltpu.async_copy(rows1, acc.at[dsts.at[a + 1]], ss1, add=True)
            return carry

        lax.fori_loop(0, NCHUNK // 2, body, 0)
        wait_g(rows0, gs0)
        pltpu.async_copy(rows0, acc.at[dsts.at[NCHUNK - 1]], ss0, add=True)
        wait_s(rows0, ss0)
        wait_s(rows1, ss1)
        plsc.subcore_barrier()

        @pl.when(sid < NS - 1)
        def _():
            pltpu.sync_copy(acc.at[pl.ds(sid * SLAB, SLAB)],
                            parts_hbm.at[cid, pl.ds(sid * SLAB, SLAB)])

        @pl.when(sid == NS - 1)
        def _():
            pltpu.sync_copy(acc.at[pl.ds(sid * SLAB, LAST_SLAB)],
                            parts_hbm.at[cid, pl.ds(sid * SLAB, LAST_SLAB)])

    return agg


_agg128 = _make_agg(H1)


R = 1000
G = N // R


def _tc1_body(x_ref, w_ref, degT_ref, xw_ref, xs_ref, dinv_ref):
    deg = degT_ref[:, 0] + degT_ref[:, 1] + 1.0
    dv = lax.rsqrt(deg)
    dinv_ref[...] = dv[:, None]
    xw = jnp.dot(x_ref[...], w_ref[...], preferred_element_type=jnp.float32)
    xw_ref[...] = xw
    xs_ref[...] = xw * dv[:, None]


def _tc1(x, w1, degT):
    return pl.pallas_call(
        _tc1_body,
        grid=(G,),
        in_specs=[
            pl.BlockSpec((R, D_IN), lambda i: (i, 0)),
            pl.BlockSpec((D_IN, H1), lambda i: (0, 0)),
            pl.BlockSpec((R, NC), lambda i: (i, 0)),
        ],
        out_specs=[
            pl.BlockSpec((R, H1), lambda i: (i, 0)),
            pl.BlockSpec((R, H1), lambda i: (i, 0)),
            pl.BlockSpec((R, 1), lambda i: (i, 0)),
        ],
        out_shape=[
            jax.ShapeDtypeStruct((N, H1), jnp.float32),
            jax.ShapeDtypeStruct((N, H1), jnp.float32),
            jax.ShapeDtypeStruct((N, 1), jnp.float32),
        ],
    )(x, w1, degT)


def _tc2_body(p_ref, dv_ref, xw_ref, b1_ref, w2_ref, hw_ref, xs2_ref):
    agg = p_ref[0] + p_ref[1]
    dv = dv_ref[...]
    h = jnp.maximum(agg * dv + xw_ref[...] * (dv * dv) + b1_ref[...], 0.0)
    hw = jnp.dot(h, w2_ref[...], preferred_element_type=jnp.float32)
    hw_ref[...] = hw
    xs2_ref[...] = jnp.concatenate(
        [hw * dv, jnp.zeros((R, H1 - H2), jnp.float32)], axis=1
    )


def _tc2(parts, dinv, xw, b1, w2):
    return pl.pallas_call(
        _tc2_body,
        grid=(G,),
        in_specs=[
            pl.BlockSpec((NC, R, H1), lambda i: (0, i, 0)),
            pl.BlockSpec((R, 1), lambda i: (i, 0)),
            pl.BlockSpec((R, H1), lambda i: (i, 0)),
            pl.BlockSpec((1, H1), lambda i: (0, 0)),
            pl.BlockSpec((H1, H2), lambda i: (0, 0)),
        ],
        out_specs=[
            pl.BlockSpec((R, H2), lambda i: (i, 0)),
            pl.BlockSpec((R, H1), lambda i: (i, 0)),
        ],
        out_shape=[
            jax.ShapeDtypeStruct((N, H2), jnp.float32),
            jax.ShapeDtypeStruct((N, H1), jnp.float32),
        ],
    )(parts, dinv, xw, b1, w2)


def _tc3_body(p_ref, dv_ref, hw_ref, b2_ref, wc_ref, bc_ref, out_ref):
    agg = (p_ref[0] + p_ref[1])[:, :H2]
    dv = dv_ref[...]
    h = jnp.maximum(agg * dv + hw_ref[...] * (dv * dv) + b2_ref[...], 0.0)
    out_ref[...] = (
        jnp.dot(h, wc_ref[...], preferred_element_type=jnp.float32) + bc_ref[...]
    )


def _tc3(parts, dinv, hw, b2, wc, bc):
    return pl.pallas_call(
        _tc3_body,
        grid=(G,),
        in_specs=[
            pl.BlockSpec((NC, R, H1), lambda i: (0, i, 0)),
            pl.BlockSpec((R, 1), lambda i: (i, 0)),
            pl.BlockSpec((R, H2), lambda i: (i, 0)),
            pl.BlockSpec((1, H2), lambda i: (0, 0)),
            pl.BlockSpec((H2, C), lambda i: (0, 0)),
            pl.BlockSpec((1, C), lambda i: (0, 0)),
        ],
        out_specs=pl.BlockSpec((R, C), lambda i: (i, 0)),
        out_shape=jax.ShapeDtypeStruct((N, C), jnp.float32),
    )(parts, dinv, hw, b2, wc, bc)


def kernel(x, edge_index, W1, b1, W2, b2, Wc, bc):
    src = edge_index[0].astype(jnp.int32)
    dst3 = edge_index[1].astype(jnp.int32).reshape(NW, NCHUNK, B)
    degp = _deg_kernel(dst3)
    xw, xs, dinv = _tc1(x, W1, degp[:, 0, :N].T)
    parts1 = _agg128(xs, src, dst3)
    hw, xs2 = _tc2(parts1, dinv, xw, b1.reshape(1, H1), W2)
    parts2 = _agg128(xs2, src, dst3)
    return _tc3(parts2, dinv, hw, b2.reshape(1, H2), Wc, bc.reshape(1, C))


# speedup vs baseline: 26.0000x; 1.0063x over previous
import functools

import jax
import jax.numpy as jnp
from jax import lax
from jax.experimental import pallas as pl
from jax.experimental.pallas import tpu as pltpu
from jax.experimental.pallas import tpu_sc as plsc

N = 10000
E = 320000
D_IN = 128
H1 = 128
H2 = 64
C = 10

NC = 2
NS = 16
NW = NC * NS
EPW = E // NW
B = 80
NCHUNK = EPW // B
SLAB = 640
LAST_SLAB = N - (NS - 1) * SLAB
NPAD = NS * SLAB
assert E % NW == 0 and EPW % B == 0 and LAST_SLAB > 0 and LAST_SLAB % B == 0
assert NCHUNK % 2 == 1

_MESH = plsc.VectorSubcoreMesh(
    core_axis_name="c", subcore_axis_name="s", num_cores=NC, num_subcores=NS
)


DEG_DEPTH = 4


@functools.partial(
    pl.kernel,
    out_type=jax.ShapeDtypeStruct((NC, 1, NPAD), jnp.float32),
    mesh=_MESH,
    scratch_types=[
        pltpu.VMEM((NCHUNK, B), jnp.int32),
        pltpu.VMEM((B,), jnp.float32),
        pltpu.VMEM((SLAB,), jnp.float32),
        pltpu.VMEM_SHARED((NPAD,), jnp.float32),
        pltpu.SemaphoreType.DMA,
    ],
)
def _deg_kernel(dst3_hbm, degp_hbm, dsts, ones_v, zeros_v, acc, sem):
    cid = lax.axis_index("c")
    sid = lax.axis_index("s")
    wid = sid * NC + cid
    one16 = jnp.ones((16,), jnp.float32)
    zero16 = jnp.zeros((16,), jnp.float32)
    for i in range(B // 16):
        ones_v[pl.ds(i * 16, 16)] = one16
    for i in range(SLAB // 16):
        zeros_v[pl.ds(i * 16, 16)] = zero16

    pltpu.sync_copy(zeros_v, acc.at[pl.ds(sid * SLAB, SLAB)])
    pltpu.sync_copy(dst3_hbm.at[wid], dsts)
    plsc.subcore_barrier()

    for k in range(DEG_DEPTH):
        pltpu.async_copy(ones_v, acc.at[dsts.at[k]], sem, add=True)

    def body(c, carry):
        pltpu.make_async_copy(ones_v, acc.at[dsts.at[0]], sem).wait()
        pltpu.async_copy(ones_v, acc.at[dsts.at[c]], sem, add=True)
        return carry

    lax.fori_loop(DEG_DEPTH, NCHUNK, body, 0)
    for k in range(DEG_DEPTH):
        pltpu.make_async_copy(ones_v, acc.at[dsts.at[0]], sem).wait()
    plsc.subcore_barrier()
    pltpu.sync_copy(acc.at[pl.ds(sid * SLAB, SLAB)],
                    degp_hbm.at[cid, 0, pl.ds(sid * SLAB, SLAB)])


def _make_agg(D):
    @functools.partial(
        pl.kernel,
        out_type=jax.ShapeDtypeStruct((NC, N, D), jnp.float32),
        mesh=_MESH,
        scratch_types=[
            pltpu.VMEM((EPW,), jnp.int32),
            pltpu.VMEM((NCHUNK, B), jnp.int32),
            pltpu.VMEM((B, D), jnp.float32),
            pltpu.VMEM((B, D), jnp.float32),
            pltpu.VMEM_SHARED((N, D), jnp.float32),
            pltpu.SemaphoreType.DMA,
            pltpu.SemaphoreType.DMA,
            pltpu.SemaphoreType.DMA,
            pltpu.SemaphoreType.DMA,
        ],
    )
    def agg(xs_hbm, src_hbm, dst3_hbm, parts_hbm,
            srcs, dsts, rows0, rows1, acc, gs0, gs1, ss0, ss1):
        cid = lax.axis_index("c")
        sid = lax.axis_index("s")
        wid = sid * NC + cid
        zero16 = jnp.zeros((16,), jnp.float32)

        def zrow(r, carry):
            for j in range(D // 16):
                rows0[r, pl.ds(j * 16, 16)] = zero16
            return carry

        lax.fori_loop(0, B, zrow, 0)
        for b in range(SLAB // B):
            if b < LAST_SLAB // B:
                pltpu.sync_copy(rows0, acc.at[pl.ds(sid * SLAB + b * B, B)])
            else:
                @pl.when(sid < NS - 1)
                def _():
                    pltpu.sync_copy(rows0, acc.at[pl.ds(sid * SLAB + b * B, B)])

        pltpu.sync_copy(src_hbm.at[pl.ds(wid * EPW, EPW)], srcs)
        pltpu.sync_copy(dst3_hbm.at[wid], dsts)
        plsc.subcore_barrier()

        def gidx(c):
            return srcs.at[pl.ds(c * B, B)]

        def wait_g(rows, sem):
            pltpu.make_async_copy(xs_hbm.at[gidx(0)], rows, sem).wait()

        def wait_s(rows, sem):
            pltpu.make_async_copy(rows, acc.at[dsts.at[0]], sem).wait()

        pltpu.async_copy(xs_hbm.at[gidx(0)], rows0, gs0)

        def body(i, carry):
            a = 2 * i
            wait_g(rows0, gs0)

            @pl.when(i > 0)
            def _():
                wait_s(rows1, ss1)
            pltpu.async_copy(xs_hbm.at[gidx(a + 1)], rows1, gs1)
            pltpu.async_copy(rows0, acc.at[dsts.at[a]], ss0, add=True)
            wait_g(rows1, gs1)
            wait_s(rows0, ss0)
            pltpu.async_copy(xs_hbm.at[gidx(a + 2)], rows0, gs0)
            pltpu.async_copy(rows1, acc.at[dsts.at[a + 1]], ss1, add=True)
            return carry

        lax.fori_loop(0, NCHUNK // 2, body, 0)
        wait_g(rows0, gs0)
        pltpu.async_copy(rows0, acc.at[dsts.at[NCHUNK - 1]], ss0, add=True)
        wait_s(rows0, ss0)
        wait_s(rows1, ss1)
        plsc.subcore_barrier()

        @pl.when(sid < NS - 1)
        def _():
            pltpu.sync_copy(acc.at[pl.ds(sid * SLAB, SLAB)],
                            parts_hbm.at[cid, pl.ds(sid * SLAB, SLAB)])

        @pl.when(sid == NS - 1)
        def _():
            pltpu.sync_copy(acc.at[pl.ds(sid * SLAB, LAST_SLAB)],
                            parts_hbm.at[cid, pl.ds(sid * SLAB, LAST_SLAB)])

    return agg


_agg128 = _make_agg(H1)


R = 1000
G = N // R


def _tc1_body(x_ref, w_ref, degT_ref, xs_ref, dinv_ref):
    deg = degT_ref[:, 0] + degT_ref[:, 1] + 1.0
    dv = lax.rsqrt(deg)
    dinv_ref[...] = dv[:, None]
    xw = jnp.dot(x_ref[...], w_ref[...], preferred_element_type=jnp.float32)
    xs_ref[...] = xw * dv[:, None]


def _tc1(x, w1, degT):
    return pl.pallas_call(
        _tc1_body,
        grid=(G,),
        in_specs=[
            pl.BlockSpec((R, D_IN), lambda i: (i, 0)),
            pl.BlockSpec((D_IN, H1), lambda i: (0, 0)),
            pl.BlockSpec((R, NC), lambda i: (i, 0)),
        ],
        out_specs=[
            pl.BlockSpec((R, H1), lambda i: (i, 0)),
            pl.BlockSpec((R, 1), lambda i: (i, 0)),
        ],
        out_shape=[
            jax.ShapeDtypeStruct((N, H1), jnp.float32),
            jax.ShapeDtypeStruct((N, 1), jnp.float32),
        ],
    )(x, w1, degT)


def _tc2_body(p_ref, dv_ref, xs_ref, b1_ref, w2_ref, xs2_ref):
    dv = dv_ref[...]
    h = jnp.maximum((p_ref[0] + p_ref[1] + xs_ref[...]) * dv + b1_ref[...], 0.0)
    hw = jnp.dot(h, w2_ref[...], preferred_element_type=jnp.float32)
    xs2_ref[...] = jnp.concatenate(
        [hw * dv, jnp.zeros((R, H1 - H2), jnp.float32)], axis=1
    )


def _tc2(parts, dinv, xs, b1, w2):
    return pl.pallas_call(
        _tc2_body,
        grid=(G,),
        in_specs=[
            pl.BlockSpec((NC, R, H1), lambda i: (0, i, 0)),
            pl.BlockSpec((R, 1), lambda i: (i, 0)),
            pl.BlockSpec((R, H1), lambda i: (i, 0)),
            pl.BlockSpec((1, H1), lambda i: (0, 0)),
            pl.BlockSpec((H1, H2), lambda i: (0, 0)),
        ],
        out_specs=pl.BlockSpec((R, H1), lambda i: (i, 0)),
        out_shape=jax.ShapeDtypeStruct((N, H1), jnp.float32),
    )(parts, dinv, xs, b1, w2)


def _tc3_body(p_ref, dv_ref, xs2_ref, b2_ref, wc_ref, bc_ref, out_ref):
    agg = (p_ref[0] + p_ref[1] + xs2_ref[...])[:, :H2]
    dv = dv_ref[...]
    h = jnp.maximum(agg * dv + b2_ref[...], 0.0)
    out_ref[...] = (
        jnp.dot(h, wc_ref[...], preferred_element_type=jnp.float32) + bc_ref[...]
    )


def _tc3(parts, dinv, xs2, b2, wc, bc):
    return pl.pallas_call(
        _tc3_body,
        grid=(G,),
        in_specs=[
            pl.BlockSpec((NC, R, H1), lambda i: (0, i, 0)),
            pl.BlockSpec((R, 1), lambda i: (i, 0)),
            pl.BlockSpec((R, H1), lambda i: (i, 0)),
            pl.BlockSpec((1, H2), lambda i: (0, 0)),
            pl.BlockSpec((H2, C), lambda i: (0, 0)),
            pl.BlockSpec((1, C), lambda i: (0, 0)),
        ],
        out_specs=pl.BlockSpec((R, C), lambda i: (i, 0)),
        out_shape=jax.ShapeDtypeStruct((N, C), jnp.float32),
    )(parts, dinv, xs2, b2, wc, bc)


def kernel(x, edge_index, W1, b1, W2, b2, Wc, bc):
    src = edge_index[0].astype(jnp.int32)
    dst3 = edge_index[1].astype(jnp.int32).reshape(NW, NCHUNK, B)
    degp = _deg_kernel(dst3)
    xs, dinv = _tc1(x, W1, degp[:, 0, :N].T)
    parts1 = _agg128(xs, src, dst3)
    xs2 = _tc2(parts1, dinv, xs, b1.reshape(1, H1), W2)
    parts2 = _agg128(xs2, src, dst3)
    return _tc3(parts2, dinv, xs2, b2.reshape(1, H2), Wc, bc.reshape(1, C))


# speedup vs baseline: 26.5055x; 1.0194x over previous
import functools

import jax
import jax.numpy as jnp
from jax import lax
from jax.experimental import pallas as pl
from jax.experimental.pallas import tpu as pltpu
from jax.experimental.pallas import tpu_sc as plsc

N = 10000
E = 320000
D_IN = 128
H1 = 128
H2 = 64
C = 10

NC = 2
NS = 16
NW = NC * NS
EPW = E // NW
B = 80
NCHUNK = EPW // B
SLAB = 640
LAST_SLAB = N - (NS - 1) * SLAB
NPAD = NS * SLAB
assert E % NW == 0 and EPW % B == 0 and LAST_SLAB > 0 and LAST_SLAB % B == 0
assert NCHUNK % 2 == 1

_MESH = plsc.VectorSubcoreMesh(
    core_axis_name="c", subcore_axis_name="s", num_cores=NC, num_subcores=NS
)


DEG_DEPTH = 4


@functools.partial(
    pl.kernel,
    out_type=jax.ShapeDtypeStruct((NC, 1, NPAD), jnp.float32),
    mesh=_MESH,
    scratch_types=[
        pltpu.VMEM((NCHUNK, B), jnp.int32),
        pltpu.VMEM((B,), jnp.float32),
        pltpu.VMEM((SLAB,), jnp.float32),
        pltpu.VMEM_SHARED((NPAD,), jnp.float32),
        pltpu.SemaphoreType.DMA,
    ],
)
def _deg_kernel(dst3_hbm, degp_hbm, dsts, ones_v, zeros_v, acc, sem):
    cid = lax.axis_index("c")
    sid = lax.axis_index("s")
    wid = sid * NC + cid
    one16 = jnp.ones((16,), jnp.float32)
    zero16 = jnp.zeros((16,), jnp.float32)
    for i in range(B // 16):
        ones_v[pl.ds(i * 16, 16)] = one16
    for i in range(SLAB // 16):
        zeros_v[pl.ds(i * 16, 16)] = zero16

    pltpu.sync_copy(zeros_v, acc.at[pl.ds(sid * SLAB, SLAB)])
    pltpu.sync_copy(dst3_hbm.at[wid], dsts)
    plsc.subcore_barrier()

    for k in range(DEG_DEPTH):
        pltpu.async_copy(ones_v, acc.at[dsts.at[k]], sem, add=True)

    def body(c, carry):
        pltpu.make_async_copy(ones_v, acc.at[dsts.at[0]], sem).wait()
        pltpu.async_copy(ones_v, acc.at[dsts.at[c]], sem, add=True)
        return carry

    lax.fori_loop(DEG_DEPTH, NCHUNK, body, 0)
    for k in range(DEG_DEPTH):
        pltpu.make_async_copy(ones_v, acc.at[dsts.at[0]], sem).wait()
    plsc.subcore_barrier()
    pltpu.sync_copy(acc.at[pl.ds(sid * SLAB, SLAB)],
                    degp_hbm.at[cid, 0, pl.ds(sid * SLAB, SLAB)])


def _make_agg(D):
    @functools.partial(
        pl.kernel,
        out_type=jax.ShapeDtypeStruct((NC, N, D), jnp.float32),
        mesh=_MESH,
        scratch_types=[
            pltpu.VMEM((EPW,), jnp.int32),
            pltpu.VMEM((NCHUNK, B), jnp.int32),
            pltpu.VMEM((B, D), jnp.float32),
            pltpu.VMEM((B, D), jnp.float32),
            pltpu.VMEM_SHARED((N, D), jnp.float32),
            pltpu.SemaphoreType.DMA,
            pltpu.SemaphoreType.DMA,
            pltpu.SemaphoreType.DMA,
            pltpu.SemaphoreType.DMA,
        ],
    )
    def agg(xs_hbm, src_hbm, dst3_hbm, parts_hbm,
            srcs, dsts, rows0, rows1, acc, gs0, gs1, ss0, ss1):
        cid = lax.axis_index("c")
        sid = lax.axis_index("s")
        wid = sid * NC + cid
        zero16 = jnp.zeros((16,), jnp.float32)

        def zrow(r, carry):
            for j in range(D // 16):
                rows0[r, pl.ds(j * 16, 16)] = zero16
            return carry

        lax.fori_loop(0, B, zrow, 0)
        for b in range(SLAB // B):
            if b < LAST_SLAB // B:
                pltpu.sync_copy(rows0, acc.at[pl.ds(sid * SLAB + b * B, B)])
            else:
                @pl.when(sid < NS - 1)
                def _():
                    pltpu.sync_copy(rows0, acc.at[pl.ds(sid * SLAB + b * B, B)])

        pltpu.sync_copy(src_hbm.at[pl.ds(wid * EPW, EPW)], srcs)
        pltpu.sync_copy(dst3_hbm.at[wid], dsts)
        plsc.subcore_barrier()

        def gidx(c):
            return srcs.at[pl.ds(c * B, B)]

        def wait_g(rows, sem):
            pltpu.make_async_copy(xs_hbm.at[gidx(0)], rows, sem).wait()

        def wait_s(rows, sem):
            pltpu.make_async_copy(rows, acc.at[dsts.at[0]], sem).wait()

        pltpu.async_copy(xs_hbm.at[gidx(0)], rows0, gs0)

        def body(i, carry):
            a = 2 * i
            wait_g(rows0, gs0)

            @pl.when(i > 0)
            def _():
                wait_s(rows1, ss1)
            pltpu.async_copy(xs_hbm.at[gidx(a + 1)], rows1, gs1)
            pltpu.async_copy(rows0, acc.at[dsts.at[a]], ss0, add=True)
            wait_g(rows1, gs1)
            wait_s(rows0, ss0)
            pltpu.async_copy(xs_hbm.at[gidx(a + 2)], rows0, gs0)
            pltpu.async_copy(rows1, acc.at[dsts.at[a + 1]], ss1, add=True)
            return carry

        lax.fori_loop(0, NCHUNK // 2, body, 0)
        wait_g(rows0, gs0)
        pltpu.async_copy(rows0, acc.at[dsts.at[NCHUNK - 1]], ss0, add=True)
        wait_s(rows0, ss0)
        wait_s(rows1, ss1)
        plsc.subcore_barrier()

        @pl.when(sid < NS - 1)
        def _():
            pltpu.sync_copy(acc.at[pl.ds(sid * SLAB, SLAB)],
                            parts_hbm.at[cid, pl.ds(sid * SLAB, SLAB)])

        @pl.when(sid == NS - 1)
        def _():
            pltpu.sync_copy(acc.at[pl.ds(sid * SLAB, LAST_SLAB)],
                            parts_hbm.at[cid, pl.ds(sid * SLAB, LAST_SLAB)])

    return agg


_agg128 = _make_agg(H1)


R = 2000
G = N // R


def _tc1_body(x_ref, w_ref, degT_ref, xs_ref, dinv_ref):
    deg = degT_ref[:, 0] + degT_ref[:, 1] + 1.0
    dv = lax.rsqrt(deg)
    dinv_ref[...] = dv[:, None]
    xw = jnp.dot(x_ref[...], w_ref[...], preferred_element_type=jnp.float32)
    xs_ref[...] = xw * dv[:, None]


def _tc1(x, w1, degT):
    return pl.pallas_call(
        _tc1_body,
        grid=(G,),
        in_specs=[
            pl.BlockSpec((R, D_IN), lambda i: (i, 0)),
            pl.BlockSpec((D_IN, H1), lambda i: (0, 0)),
            pl.BlockSpec((R, NC), lambda i: (i, 0)),
        ],
        out_specs=[
            pl.BlockSpec((R, H1), lambda i: (i, 0)),
            pl.BlockSpec((R, 1), lambda i: (i, 0)),
        ],
        out_shape=[
            jax.ShapeDtypeStruct((N, H1), jnp.float32),
            jax.ShapeDtypeStruct((N, 1), jnp.float32),
        ],
    )(x, w1, degT)


def _tc2_body(p_ref, dv_ref, xs_ref, b1_ref, w2_ref, xs2_ref):
    dv = dv_ref[...]
    h = jnp.maximum((p_ref[0] + p_ref[1] + xs_ref[...]) * dv + b1_ref[...], 0.0)
    hw = jnp.dot(h, w2_ref[...], preferred_element_type=jnp.float32)
    xs2_ref[...] = jnp.concatenate(
        [hw * dv, jnp.zeros((R, H1 - H2), jnp.float32)], axis=1
    )


def _tc2(parts, dinv, xs, b1, w2):
    return pl.pallas_call(
        _tc2_body,
        grid=(G,),
        in_specs=[
            pl.BlockSpec((NC, R, H1), lambda i: (0, i, 0)),
            pl.BlockSpec((R, 1), lambda i: (i, 0)),
            pl.BlockSpec((R, H1), lambda i: (i, 0)),
            pl.BlockSpec((1, H1), lambda i: (0, 0)),
            pl.BlockSpec((H1, H2), lambda i: (0, 0)),
        ],
        out_specs=pl.BlockSpec((R, H1), lambda i: (i, 0)),
        out_shape=jax.ShapeDtypeStruct((N, H1), jnp.float32),
    )(parts, dinv, xs, b1, w2)


def _tc3_body(p_ref, dv_ref, xs2_ref, b2_ref, wc_ref, bc_ref, out_ref):
    agg = (p_ref[0] + p_ref[1] + xs2_ref[...])[:, :H2]
    dv = dv_ref[...]
    h = jnp.maximum(agg * dv + b2_ref[...], 0.0)
    out_ref[...] = (
        jnp.dot(h, wc_ref[...], preferred_element_type=jnp.float32) + bc_ref[...]
    )


def _tc3(parts, dinv, xs2, b2, wc, bc):
    return pl.pallas_call(
        _tc3_body,
        grid=(G,),
        in_specs=[
            pl.BlockSpec((NC, R, H1), lambda i: (0, i, 0)),
            pl.BlockSpec((R, 1), lambda i: (i, 0)),
            pl.BlockSpec((R, H1), lambda i: (i, 0)),
            pl.BlockSpec((1, H2), lambda i: (0, 0)),
            pl.BlockSpec((H2, C), lambda i: (0, 0)),
            pl.BlockSpec((1, C), lambda i: (0, 0)),
        ],
        out_specs=pl.BlockSpec((R, C), lambda i: (i, 0)),
        out_shape=jax.ShapeDtypeStruct((N, C), jnp.float32),
    )(parts, dinv, xs2, b2, wc, bc)


def kernel(x, edge_index, W1, b1, W2, b2, Wc, bc):
    src = edge_index[0].astype(jnp.int32)
    dst3 = edge_index[1].astype(jnp.int32).reshape(NW, NCHUNK, B)
    degp = _deg_kernel(dst3)
    xs, dinv = _tc1(x, W1, degp[:, 0, :N].T)
    parts1 = _agg128(xs, src, dst3)
    xs2 = _tc2(parts1, dinv, xs, b1.reshape(1, H1), W2)
    parts2 = _agg128(xs2, src, dst3)
    return _tc3(parts2, dinv, xs2, b2.reshape(1, H2), Wc, bc.reshape(1, C))


# speedup vs baseline: 26.7116x; 1.0078x over previous
import functools

import jax
import jax.numpy as jnp
from jax import lax
from jax.experimental import pallas as pl
from jax.experimental.pallas import tpu as pltpu
from jax.experimental.pallas import tpu_sc as plsc

N = 10000
E = 320000
D_IN = 128
H1 = 128
H2 = 64
C = 10

NC = 2
NS = 16
NW = NC * NS
EPW = E // NW
B = 80
NCHUNK = EPW // B
SLAB = 640
LAST_SLAB = N - (NS - 1) * SLAB
NPAD = NS * SLAB
assert E % NW == 0 and EPW % B == 0 and LAST_SLAB > 0 and LAST_SLAB % B == 0
assert NCHUNK % 2 == 1

_MESH = plsc.VectorSubcoreMesh(
    core_axis_name="c", subcore_axis_name="s", num_cores=NC, num_subcores=NS
)


DEG_DEPTH = 4


@functools.partial(
    pl.kernel,
    out_type=jax.ShapeDtypeStruct((NC, 1, NPAD), jnp.float32),
    mesh=_MESH,
    scratch_types=[
        pltpu.VMEM((NCHUNK, B), jnp.int32),
        pltpu.VMEM((B,), jnp.float32),
        pltpu.VMEM((SLAB,), jnp.float32),
        pltpu.VMEM_SHARED((NPAD,), jnp.float32),
        pltpu.SemaphoreType.DMA,
    ],
)
def _deg_kernel(dst3_hbm, degp_hbm, dsts, ones_v, zeros_v, acc, sem):
    cid = lax.axis_index("c")
    sid = lax.axis_index("s")
    wid = sid * NC + cid
    one16 = jnp.ones((16,), jnp.float32)
    zero16 = jnp.zeros((16,), jnp.float32)
    for i in range(B // 16):
        ones_v[pl.ds(i * 16, 16)] = one16
    for i in range(SLAB // 16):
        zeros_v[pl.ds(i * 16, 16)] = zero16

    pltpu.sync_copy(zeros_v, acc.at[pl.ds(sid * SLAB, SLAB)])
    pltpu.sync_copy(dst3_hbm.at[wid], dsts)
    plsc.subcore_barrier()

    for k in range(DEG_DEPTH):
        pltpu.async_copy(ones_v, acc.at[dsts.at[k]], sem, add=True)

    def body(c, carry):
        pltpu.make_async_copy(ones_v, acc.at[dsts.at[0]], sem).wait()
        pltpu.async_copy(ones_v, acc.at[dsts.at[c]], sem, add=True)
        return carry

    lax.fori_loop(DEG_DEPTH, NCHUNK, body, 0)
    for k in range(DEG_DEPTH):
        pltpu.make_async_copy(ones_v, acc.at[dsts.at[0]], sem).wait()
    plsc.subcore_barrier()
    pltpu.sync_copy(acc.at[pl.ds(sid * SLAB, SLAB)],
                    degp_hbm.at[cid, 0, pl.ds(sid * SLAB, SLAB)])


def _make_agg(D):
    @functools.partial(
        pl.kernel,
        out_type=jax.ShapeDtypeStruct((NC, N, D), jnp.float32),
        mesh=_MESH,
        scratch_types=[
            pltpu.VMEM((EPW,), jnp.int32),
            pltpu.VMEM((NCHUNK, B), jnp.int32),
            pltpu.VMEM((B, D), jnp.float32),
            pltpu.VMEM((B, D), jnp.float32),
            pltpu.VMEM_SHARED((N, D), jnp.float32),
            pltpu.SemaphoreType.DMA,
            pltpu.SemaphoreType.DMA,
            pltpu.SemaphoreType.DMA,
            pltpu.SemaphoreType.DMA,
        ],
    )
    def agg(xs_hbm, src_hbm, dst3_hbm, parts_hbm,
            srcs, dsts, rows0, rows1, acc, gs0, gs1, ss0, ss1):
        cid = lax.axis_index("c")
        sid = lax.axis_index("s")
        wid = sid * NC + cid
        zero16 = jnp.zeros((16,), jnp.float32)

        def zrow(r, carry):
            for j in range(D // 16):
                rows0[r, pl.ds(j * 16, 16)] = zero16
            return carry

        lax.fori_loop(0, B, zrow, 0)
        for b in range(SLAB // B):
            if b < LAST_SLAB // B:
                pltpu.sync_copy(rows0, acc.at[pl.ds(sid * SLAB + b * B, B)])
            else:
                @pl.when(sid < NS - 1)
                def _():
                    pltpu.sync_copy(rows0, acc.at[pl.ds(sid * SLAB + b * B, B)])

        pltpu.sync_copy(src_hbm.at[pl.ds(wid * EPW, EPW)], srcs)
        pltpu.sync_copy(dst3_hbm.at[wid], dsts)
        plsc.subcore_barrier()

        def gidx(c):
            return srcs.at[pl.ds(c * B, B)]

        def wait_g(rows, sem):
            pltpu.make_async_copy(xs_hbm.at[gidx(0)], rows, sem).wait()

        def wait_s(rows, sem):
            pltpu.make_async_copy(rows, acc.at[dsts.at[0]], sem).wait()

        pltpu.async_copy(xs_hbm.at[gidx(0)], rows0, gs0)

        def body(i, carry):
            a = 2 * i
            wait_g(rows0, gs0)

            @pl.when(i > 0)
            def _():
                wait_s(rows1, ss1)
            pltpu.async_copy(xs_hbm.at[gidx(a + 1)], rows1, gs1)
            pltpu.async_copy(rows0, acc.at[dsts.at[a]], ss0, add=True)
            wait_g(rows1, gs1)
            wait_s(rows0, ss0)
            pltpu.async_copy(xs_hbm.at[gidx(a + 2)], rows0, gs0)
            pltpu.async_copy(rows1, acc.at[dsts.at[a + 1]], ss1, add=True)
            return carry

        lax.fori_loop(0, NCHUNK // 2, body, 0)
        wait_g(rows0, gs0)
        pltpu.async_copy(rows0, acc.at[dsts.at[NCHUNK - 1]], ss0, add=True)
        wait_s(rows0, ss0)
        wait_s(rows1, ss1)
        plsc.subcore_barrier()

        @pl.when(sid < NS - 1)
        def _():
            pltpu.sync_copy(acc.at[pl.ds(sid * SLAB, SLAB)],
                            parts_hbm.at[cid, pl.ds(sid * SLAB, SLAB)])

        @pl.when(sid == NS - 1)
        def _():
            pltpu.sync_copy(acc.at[pl.ds(sid * SLAB, LAST_SLAB)],
                            parts_hbm.at[cid, pl.ds(sid * SLAB, LAST_SLAB)])

    return agg


_agg128 = _make_agg(H1)


R = 5000
G = N // R


def _tc1_body(x_ref, w_ref, degT_ref, xs_ref, dinv_ref):
    deg = degT_ref[:, 0] + degT_ref[:, 1] + 1.0
    dv = lax.rsqrt(deg)
    dinv_ref[...] = dv[:, None]
    xw = jnp.dot(x_ref[...], w_ref[...], preferred_element_type=jnp.float32)
    xs_ref[...] = xw * dv[:, None]


def _tc1(x, w1, degT):
    return pl.pallas_call(
        _tc1_body,
        grid=(G,),
        in_specs=[
            pl.BlockSpec((R, D_IN), lambda i: (i, 0)),
            pl.BlockSpec((D_IN, H1), lambda i: (0, 0)),
            pl.BlockSpec((R, NC), lambda i: (i, 0)),
        ],
        out_specs=[
            pl.BlockSpec((R, H1), lambda i: (i, 0)),
            pl.BlockSpec((R, 1), lambda i: (i, 0)),
        ],
        out_shape=[
            jax.ShapeDtypeStruct((N, H1), jnp.float32),
            jax.ShapeDtypeStruct((N, 1), jnp.float32),
        ],
    )(x, w1, degT)


def _tc2_body(p_ref, dv_ref, xs_ref, b1_ref, w2_ref, xs2_ref):
    dv = dv_ref[...]
    h = jnp.maximum((p_ref[0] + p_ref[1] + xs_ref[...]) * dv + b1_ref[...], 0.0)
    hw = jnp.dot(h, w2_ref[...], preferred_element_type=jnp.float32)
    xs2_ref[...] = jnp.concatenate(
        [hw * dv, jnp.zeros((R, H1 - H2), jnp.float32)], axis=1
    )


def _tc2(parts, dinv, xs, b1, w2):
    return pl.pallas_call(
        _tc2_body,
        grid=(G,),
        in_specs=[
            pl.BlockSpec((NC, R, H1), lambda i: (0, i, 0)),
            pl.BlockSpec((R, 1), lambda i: (i, 0)),
            pl.BlockSpec((R, H1), lambda i: (i, 0)),
            pl.BlockSpec((1, H1), lambda i: (0, 0)),
            pl.BlockSpec((H1, H2), lambda i: (0, 0)),
        ],
        out_specs=pl.BlockSpec((R, H1), lambda i: (i, 0)),
        out_shape=jax.ShapeDtypeStruct((N, H1), jnp.float32),
    )(parts, dinv, xs, b1, w2)


def _tc3_body(p_ref, dv_ref, xs2_ref, b2_ref, wc_ref, bc_ref, out_ref):
    agg = (p_ref[0] + p_ref[1] + xs2_ref[...])[:, :H2]
    dv = dv_ref[...]
    h = jnp.maximum(agg * dv + b2_ref[...], 0.0)
    out_ref[...] = (
        jnp.dot(h, wc_ref[...], preferred_element_type=jnp.float32) + bc_ref[...]
    )


def _tc3(parts, dinv, xs2, b2, wc, bc):
    return pl.pallas_call(
        _tc3_body,
        grid=(G,),
        in_specs=[
            pl.BlockSpec((NC, R, H1), lambda i: (0, i, 0)),
            pl.BlockSpec((R, 1), lambda i: (i, 0)),
            pl.BlockSpec((R, H1), lambda i: (i, 0)),
            pl.BlockSpec((1, H2), lambda i: (0, 0)),
            pl.BlockSpec((H2, C), lambda i: (0, 0)),
            pl.BlockSpec((1, C), lambda i: (0, 0)),
        ],
        out_specs=pl.BlockSpec((R, C), lambda i: (i, 0)),
        out_shape=jax.ShapeDtypeStruct((N, C), jnp.float32),
    )(parts, dinv, xs2, b2, wc, bc)


def kernel(x, edge_index, W1, b1, W2, b2, Wc, bc):
    src = edge_index[0].astype(jnp.int32)
    dst3 = edge_index[1].astype(jnp.int32).reshape(NW, NCHUNK, B)
    degp = _deg_kernel(dst3)
    xs, dinv = _tc1(x, W1, degp[:, 0, :N].T)
    parts1 = _agg128(xs, src, dst3)
    xs2 = _tc2(parts1, dinv, xs, b1.reshape(1, H1), W2)
    parts2 = _agg128(xs2, src, dst3)
    return _tc3(parts2, dinv, xs2, b2.reshape(1, H2), Wc, bc.reshape(1, C))
